# baseline (speedup 1.0000x reference)
import functools

import jax
import jax.numpy as jnp
from jax import lax
from jax.experimental import pallas as pl
from jax.experimental.pallas import tpu as pltpu
from jax.experimental.pallas import tpu_sc as plsc

_FEATURE_DIM = 512
_TOP_K = 50
_ALPHA = 0.01
_TEMPERATURE = 0.1
_OLD_W = 1.0
_NEW_W = 0.3

_NC = 2
_NS = 16
_NW = _NC * _NS
_LANES = 16
_CHUNK = 64


def _vsort(x, desc=False):
    k, _ = plsc.sort_key_val(x, x, descending=desc)
    return k


def _sort_bitonic(x, desc):
    m = len(x)
    if m == 1:
        return [_vsort(x[0], desc)]
    half = m // 2
    los = [jnp.minimum(x[j], x[j + half]) for j in range(half)]
    his = [jnp.maximum(x[j], x[j + half]) for j in range(half)]
    if desc:
        return _sort_bitonic(his, True) + _sort_bitonic(los, True)
    return _sort_bitonic(los, False) + _sort_bitonic(his, False)


def _merge(a, b, desc, top_only):
    m = len(a)
    los = [jnp.minimum(a[j], b[j]) for j in range(m)]
    his = [jnp.maximum(a[j], b[j]) for j in range(m)]
    if top_only:
        return _sort_bitonic(his, desc)
    if desc:
        return _sort_bitonic(his, True) + _sort_bitonic(los, True)
    return _sort_bitonic(los, False) + _sort_bitonic(his, False)


def _row_threshold(buf, i):
    lvl = [[_vsort(jnp.abs(buf[i, pl.ds(_LANES * q, _LANES)]), q % 2 == 1)]
           for q in range(_FEATURE_DIM // _LANES)]
    lvl = [_merge(lvl[2 * a], lvl[2 * a + 1], a % 2 == 1, False)
           for a in range(16)]
    lvl = [_merge(lvl[2 * a], lvl[2 * a + 1], a % 2 == 1, False)
           for a in range(8)]
    lvl = [_merge(lvl[2 * a], lvl[2 * a + 1], a % 2 == 1, True)
           for a in range(4)]
    lvl = [_merge(lvl[2 * a], lvl[2 * a + 1], a % 2 == 1, True)
           for a in range(2)]
    his = [jnp.maximum(lvl[0][j], lvl[1][j]) for j in range(4)]
    lo2 = [jnp.minimum(his[0], his[2]), jnp.minimum(his[1], his[3])]
    lo1 = jnp.minimum(lo2[0], lo2[1])
    return _vsort(lo1)


def _sc_body(s_hbm, t_hbm, ths_hbm, tht_hbm, buf_a, buf_b, thr_buf, low_buf,
             sem_a, sem_b, *, row0, nrows):
    wid = lax.axis_index("s") * _NC + lax.axis_index("c")
    rows_per_w = nrows // _NW
    base = row0 + wid * rows_per_w
    obase = wid * rows_per_w
    nchunks = rows_per_w // _CHUNK
    lane = lax.iota(jnp.int32, _LANES)

    idx14 = jnp.full((_LANES,), 14, jnp.int32)

    def compute_chunk(buf, dst, off):
        @plsc.parallel_loop(0, _CHUNK, 1, unroll=1)
        def _(j):
            low_buf[j, :] = _row_threshold(buf, j)

        def group_body(g, _):
            goff = pl.multiple_of(g * _LANES, _LANES)
            th = plsc.load_gather(low_buf.at[pl.ds(goff, _LANES)],
                                  [lane, idx14])
            thr_buf[pl.ds(goff, _LANES)] = th
            return 0

        lax.fori_loop(0, _CHUNK // _LANES, group_body, 0)
        pltpu.sync_copy(thr_buf, dst.at[pl.ds(off, _CHUNK)])

    for src, dst in ((s_hbm, ths_hbm), (t_hbm, tht_hbm)):
        pltpu.async_copy(src.at[pl.ds(pl.multiple_of(base, _CHUNK), _CHUNK)],
                         buf_a, sem_a)

        def pair_body(c2, _, src=src, dst=dst):
            for b, (bf, sm, obf, osm) in enumerate(
                    ((buf_a, sem_a, buf_b, sem_b),
                     (buf_b, sem_b, buf_a, sem_a))):
                chunk = c2 * 2 + b
                pltpu.make_async_copy(src.at[pl.ds(0, _CHUNK)], bf, sm).wait()

                @pl.when(chunk < nchunks - 1)
                def _(chunk=chunk, obf=obf, osm=osm, src=src):
                    noff = pl.multiple_of(base + (chunk + 1) * _CHUNK, _CHUNK)
                    pltpu.async_copy(src.at[pl.ds(noff, _CHUNK)], obf, osm)

                off = pl.multiple_of(obase + chunk * _CHUNK, _CHUNK)
                compute_chunk(bf, dst, off)
            return 0

        lax.fori_loop(0, nchunks // 2, pair_body, 0)


def _sc_thresholds(student_feats, teacher_feats, row0, nrows):
    mesh = plsc.VectorSubcoreMesh(core_axis_name="c", subcore_axis_name="s")
    fn = functools.partial(
        pl.kernel,
        mesh=mesh,
        out_type=[
            jax.ShapeDtypeStruct((nrows,), jnp.float32),
            jax.ShapeDtypeStruct((nrows,), jnp.float32),
        ],
        scratch_types=[
            pltpu.VMEM((_CHUNK, _FEATURE_DIM), jnp.float32),
            pltpu.VMEM((_CHUNK, _FEATURE_DIM), jnp.float32),
            pltpu.VMEM((_CHUNK,), jnp.float32),
            pltpu.VMEM((_CHUNK, _LANES), jnp.float32),
            pltpu.SemaphoreType.DMA,
            pltpu.SemaphoreType.DMA,
        ],
        compiler_params=pltpu.CompilerParams(needs_layout_passes=False),
    )(functools.partial(_sc_body, row0=row0, nrows=nrows))
    return fn(student_feats, teacher_feats)


def _tc_body(s_ref, t_ref, ths_ref, tht_ref, w_ref, num_ref, den_ref):
    i = pl.program_id(0)
    s = s_ref[...]
    t = t_ref[...]
    fs = jnp.where(jnp.abs(s) >= ths_ref[...], 1.0, _ALPHA)
    ft = jnp.where(jnp.abs(t) >= tht_ref[...], 1.0, _ALPHA)
    sm = s * fs
    tm = t * ft
    ones = jnp.ones((s.shape[1], 1), jnp.float32)
    dot = jnp.dot(sm * tm, ones, preferred_element_type=jnp.float32)
    ss = jnp.dot(sm * sm, ones, preferred_element_type=jnp.float32)
    tt = jnp.dot(tm * tm, ones, preferred_element_type=jnp.float32)
    cos = dot / ((jnp.sqrt(ss) + 1e-8) * (jnp.sqrt(tt) + 1e-8))
    per = (1.0 - cos) / _TEMPERATURE
    w = w_ref[0]
    pnum = jnp.dot(w, per, preferred_element_type=jnp.float32)
    pden = jnp.sum(w, axis=1, keepdims=True)

    @pl.when(i == 0)
    def _():
        num_ref[...] = pnum
        den_ref[...] = pden

    @pl.when(i > 0)
    def _():
        num_ref[...] += pnum
        den_ref[...] += pden


def _tc_partial(student_feats, teacher_feats, ths, tht, w3, row0, nrows, r):
    d = student_feats.shape[1]
    g = nrows // r
    g0 = row0 // r
    return pl.pallas_call(
        _tc_body,
        grid=(g,),
        in_specs=[
            pl.BlockSpec((r, d), lambda i: (i + g0, 0)),
            pl.BlockSpec((r, d), lambda i: (i + g0, 0)),
            pl.BlockSpec((r, 1), lambda i: (i, 0)),
            pl.BlockSpec((r, 1), lambda i: (i, 0)),
            pl.BlockSpec((1, 1, r), lambda i: (i + g0, 0, 0)),
        ],
        out_specs=[
            pl.BlockSpec((1, 1), lambda i: (0, 0)),
            pl.BlockSpec((1, 1), lambda i: (0, 0)),
        ],
        out_shape=[
            jax.ShapeDtypeStruct((1, 1), jnp.float32),
            jax.ShapeDtypeStruct((1, 1), jnp.float32),
        ],
    )(student_feats, teacher_feats, ths.reshape(nrows, 1),
      tht.reshape(nrows, 1), w3)


def kernel(student_feats, teacher_feats, targets, num_old_classes):
    b, d = student_feats.shape
    r = 4096
    w = jnp.where(targets < num_old_classes, _OLD_W, _NEW_W).astype(jnp.float32)
    w3 = w.reshape(b // r, 1, r)
    ths, tht = _sc_thresholds(student_feats, teacher_feats, 0, b)
    num, den = _tc_partial(student_feats, teacher_feats, ths, tht, w3,
                           0, b, r)
    return (num[0, 0] / (den[0, 0] + 1e-8)).astype(jnp.float32)


# speedup vs baseline: 1.0058x; 1.0058x over previous
import functools

import jax
import jax.numpy as jnp
from jax import lax
from jax.experimental import pallas as pl
from jax.experimental.pallas import tpu as pltpu
from jax.experimental.pallas import tpu_sc as plsc

_FEATURE_DIM = 512
_TOP_K = 50
_ALPHA = 0.01
_TEMPERATURE = 0.1
_OLD_W = 1.0
_NEW_W = 0.3

_NC = 2
_NS = 16
_NW = _NC * _NS
_LANES = 16
_CHUNK = 64


def _vsort(x, desc=False):
    k, _ = plsc.sort_key_val(x, x, descending=desc)
    return k


def _sort_bitonic(x, desc):
    m = len(x)
    if m == 1:
        return [_vsort(x[0], desc)]
    half = m // 2
    los = [jnp.minimum(x[j], x[j + half]) for j in range(half)]
    his = [jnp.maximum(x[j], x[j + half]) for j in range(half)]
    if desc:
        return _sort_bitonic(his, True) + _sort_bitonic(los, True)
    return _sort_bitonic(los, False) + _sort_bitonic(his, False)


def _merge(a, b, desc, top_only):
    m = len(a)
    los = [jnp.minimum(a[j], b[j]) for j in range(m)]
    his = [jnp.maximum(a[j], b[j]) for j in range(m)]
    if top_only:
        return _sort_bitonic(his, desc)
    if desc:
        return _sort_bitonic(his, True) + _sort_bitonic(los, True)
    return _sort_bitonic(los, False) + _sort_bitonic(his, False)


def _row_threshold(buf, i):
    lvl = [[_vsort(jnp.abs(buf[i, pl.ds(_LANES * q, _LANES)]), q % 2 == 1)]
           for q in range(_FEATURE_DIM // _LANES)]
    lvl = [_merge(lvl[2 * a], lvl[2 * a + 1], a % 2 == 1, False)
           for a in range(16)]
    lvl = [_merge(lvl[2 * a], lvl[2 * a + 1], a % 2 == 1, False)
           for a in range(8)]
    lvl = [_merge(lvl[2 * a], lvl[2 * a + 1], a % 2 == 1, True)
           for a in range(4)]
    lvl = [_merge(lvl[2 * a], lvl[2 * a + 1], a % 2 == 1, True)
           for a in range(2)]
    his = [jnp.maximum(lvl[0][j], lvl[1][j]) for j in range(4)]
    lo2 = [jnp.minimum(his[0], his[2]), jnp.minimum(his[1], his[3])]
    lo1 = jnp.minimum(lo2[0], lo2[1])
    return _vsort(lo1)


def _sc_body(s_hbm, t_hbm, ths_hbm, tht_hbm, buf_a, buf_b, thr_buf, low_buf,
             sem_a, sem_b, *, row0, nrows):
    wid = lax.axis_index("s") * _NC + lax.axis_index("c")
    rows_per_w = nrows // _NW
    base = row0 + wid * rows_per_w
    obase = wid * rows_per_w
    nchunks = rows_per_w // _CHUNK
    lane = lax.iota(jnp.int32, _LANES)

    idx14 = jnp.full((_LANES,), 14, jnp.int32)

    def compute_chunk(buf, dst, off):
        @plsc.parallel_loop(0, _CHUNK, 1, unroll=1)
        def _(j):
            low_buf[j, :] = _row_threshold(buf, j)

        def group_body(g, _):
            goff = pl.multiple_of(g * _LANES, _LANES)
            th = plsc.load_gather(low_buf.at[pl.ds(goff, _LANES)],
                                  [lane, idx14])
            thr_buf[pl.ds(goff, _LANES)] = th
            return 0

        lax.fori_loop(0, _CHUNK // _LANES, group_body, 0)
        pltpu.sync_copy(thr_buf, dst.at[pl.ds(off, _CHUNK)])

    for src, dst in ((s_hbm, ths_hbm), (t_hbm, tht_hbm)):
        pltpu.async_copy(src.at[pl.ds(pl.multiple_of(base, _CHUNK), _CHUNK)],
                         buf_a, sem_a)

        def pair_body(c2, _, src=src, dst=dst):
            for b, (bf, sm, obf, osm) in enumerate(
                    ((buf_a, sem_a, buf_b, sem_b),
                     (buf_b, sem_b, buf_a, sem_a))):
                chunk = c2 * 2 + b
                pltpu.make_async_copy(src.at[pl.ds(0, _CHUNK)], bf, sm).wait()

                @pl.when(chunk < nchunks - 1)
                def _(chunk=chunk, obf=obf, osm=osm, src=src):
                    noff = pl.multiple_of(base + (chunk + 1) * _CHUNK, _CHUNK)
                    pltpu.async_copy(src.at[pl.ds(noff, _CHUNK)], obf, osm)

                off = pl.multiple_of(obase + chunk * _CHUNK, _CHUNK)
                compute_chunk(bf, dst, off)
            return 0

        lax.fori_loop(0, nchunks // 2, pair_body, 0)


def _sc_thresholds(student_feats, teacher_feats, row0, nrows):
    mesh = plsc.VectorSubcoreMesh(core_axis_name="c", subcore_axis_name="s")
    fn = functools.partial(
        pl.kernel,
        mesh=mesh,
        out_type=[
            jax.ShapeDtypeStruct((nrows,), jnp.float32),
            jax.ShapeDtypeStruct((nrows,), jnp.float32),
        ],
        scratch_types=[
            pltpu.VMEM((_CHUNK, _FEATURE_DIM), jnp.float32),
            pltpu.VMEM((_CHUNK, _FEATURE_DIM), jnp.float32),
            pltpu.VMEM((_CHUNK,), jnp.float32),
            pltpu.VMEM((_CHUNK, _LANES), jnp.float32),
            pltpu.SemaphoreType.DMA,
            pltpu.SemaphoreType.DMA,
        ],
        compiler_params=pltpu.CompilerParams(needs_layout_passes=False),
    )(functools.partial(_sc_body, row0=row0, nrows=nrows))
    return fn(student_feats, teacher_feats)


def _tc_body(s_ref, t_ref, ths_ref, tht_ref, w_ref, num_ref, den_ref):
    i = pl.program_id(0)
    s = s_ref[...]
    t = t_ref[...]
    fs = jnp.where(jnp.abs(s) >= ths_ref[...], 1.0, _ALPHA)
    ft = jnp.where(jnp.abs(t) >= tht_ref[...], 1.0, _ALPHA)
    sm = s * fs
    tm = t * ft
    ones = jnp.ones((s.shape[1], 1), jnp.float32)
    dot = jnp.dot(sm * tm, ones, preferred_element_type=jnp.float32)
    ss = jnp.dot(sm * sm, ones, preferred_element_type=jnp.float32)
    tt = jnp.dot(tm * tm, ones, preferred_element_type=jnp.float32)
    cos = dot / ((jnp.sqrt(ss) + 1e-8) * (jnp.sqrt(tt) + 1e-8))
    per = (1.0 - cos) / _TEMPERATURE
    w = w_ref[0]
    pnum = jnp.dot(w, per, preferred_element_type=jnp.float32)
    pden = jnp.sum(w, axis=1, keepdims=True)

    @pl.when(i == 0)
    def _():
        num_ref[...] = pnum
        den_ref[...] = pden

    @pl.when(i > 0)
    def _():
        num_ref[...] += pnum
        den_ref[...] += pden


def _tc_partial(student_feats, teacher_feats, ths, tht, w3, row0, nrows, r):
    d = student_feats.shape[1]
    g = nrows // r
    g0 = row0 // r
    return pl.pallas_call(
        _tc_body,
        grid=(g,),
        in_specs=[
            pl.BlockSpec((r, d), lambda i: (i + g0, 0)),
            pl.BlockSpec((r, d), lambda i: (i + g0, 0)),
            pl.BlockSpec((r, 1), lambda i: (i, 0)),
            pl.BlockSpec((r, 1), lambda i: (i, 0)),
            pl.BlockSpec((1, 1, r), lambda i: (i + g0, 0, 0)),
        ],
        out_specs=[
            pl.BlockSpec((1, 1), lambda i: (0, 0)),
            pl.BlockSpec((1, 1), lambda i: (0, 0)),
        ],
        out_shape=[
            jax.ShapeDtypeStruct((1, 1), jnp.float32),
            jax.ShapeDtypeStruct((1, 1), jnp.float32),
        ],
    )(student_feats, teacher_feats, ths.reshape(nrows, 1),
      tht.reshape(nrows, 1), w3)


def kernel(student_feats, teacher_feats, targets, num_old_classes):
    b, d = student_feats.shape
    r = 2048
    w = jnp.where(targets < num_old_classes, _OLD_W, _NEW_W).astype(jnp.float32)
    w3 = w.reshape(b // r, 1, r)
    ths, tht = _sc_thresholds(student_feats, teacher_feats, 0, b)
    num, den = _tc_partial(student_feats, teacher_feats, ths, tht, w3,
                           0, b, r)
    return (num[0, 0] / (den[0, 0] + 1e-8)).astype(jnp.float32)
